# Initial kernel scaffold; baseline (speedup 1.0000x reference)
import jax
import jax.numpy as jnp
from jax.experimental import pallas as pl


def kernel(x, node_pos, W_in, b_in, Wl1, bl1, Wl2, bl2, Wg1, bg1, Wg2, bg2, Wd, bd, Wm, bm, gamma, beta, edge_index):
    raise NotImplementedError("write your pallas kernel here")


# speedup vs baseline: 1.0055x; 1.0055x over previous
import functools
from math import pi as PI

import jax
import jax.numpy as jnp
import numpy as np
from jax import lax
from jax.experimental import pallas as pl
from jax.experimental.pallas import tpu as pltpu
from jax.experimental.pallas import tpu_sc as plsc

_N = 10000
_E = 320000
_D = 128
_MID = 64
_CUT = 8.0
_EPS = 1e-10

_EBLK = 2560
_NBLK = 1000

_C0 = 0.5 / np.sqrt(PI)
_C1 = np.sqrt(3.0 / (4.0 * PI))


def _dot3(a, b):
    return a[0] * b[0] + a[1] * b[1] + a[2] * b[2]


def _cross3(a, b):
    return (a[1] * b[2] - a[2] * b[1],
            a[2] * b[0] - a[0] * b[2],
            a[0] * b[1] - a[1] * b[0])


def _safe_atan2(b, a):
    guard = (a * a + b * b) < 1e-18
    a = jnp.where(guard, 1.0, a)
    b = jnp.where(guard, 0.0, b)
    return jnp.arctan2(b, a)


def _edge_kernel(geo_ref, nemb_ref, Wl1_ref, Wg1_ref, Wloc_ref, Wglb_ref,
                 beff_ref, out_ref):
    g = geo_ref[...]
    v = (g[0], g[1], g[2])
    f = (g[3], g[4], g[5])
    r = (g[6], g[7], g[8])
    jr = (g[9], g[10], g[11])
    d = g[12]
    s_in = g[13]
    nv = (-v[0], -v[1], -v[2])

    cvf = _cross3(nv, f)
    a_t = _dot3(nv, f)
    b_t = jnp.sqrt(_dot3(cvf, cvf) + 1e-12)
    theta = _safe_atan2(b_t, a_t)

    p2 = _cross3(nv, r)
    a_p = _dot3(cvf, p2)
    b_p = _dot3(_cross3(cvf, p2), v) / d
    phi = _safe_atan2(b_p, a_p)
    phi = jnp.where(phi < 0, phi + PI, phi)

    q1 = _cross3(v, jr)
    q2 = _cross3(v, r)
    a_u = _dot3(q1, q2)
    b_u = _dot3(_cross3(q1, q2), v) / d
    tau = _safe_atan2(b_u, a_u)
    tau = jnp.where(tau < 0, tau + 2.0 * PI, tau)

    pre = np.sqrt(2.0 / _CUT).astype(np.float32)
    inv_d = pre / d
    rb = [jnp.sin((n * PI / _CUT) * d) * inv_d for n in (1.0, 2.0, 3.0)]
    st, ct = jnp.sin(theta), jnp.cos(theta)
    sp, cp = jnp.sin(phi), jnp.cos(phi)
    y = [jnp.full_like(theta, _C0), _C1 * st * sp, _C1 * ct, _C1 * st * cp]
    sph = [jnp.ones_like(tau), jnp.cos(tau)]

    hl = jnp.zeros((g.shape[1], _MID), jnp.float32)
    for n in range(3):
        for m in range(4):
            feat = rb[n] * y[m]
            hl = hl + feat[:, None] * Wl1_ref[n * 4 + m, :][None, :]
    hl = hl * jax.nn.sigmoid(hl)

    hg = jnp.zeros((g.shape[1], _MID), jnp.float32)
    for n in range(3):
        for m in range(2):
            feat = rb[n] * sph[m]
            hg = hg + feat[:, None] * Wg1_ref[n * 2 + m, :][None, :]
    hg = hg * jax.nn.sigmoid(hg)

    edge_feat = (jnp.dot(hl, Wloc_ref[...], preferred_element_type=jnp.float32)
                 + jnp.dot(hg, Wglb_ref[...], preferred_element_type=jnp.float32)
                 + beff_ref[...])
    msg = (edge_feat + nemb_ref[...] + d[:, None]) * s_in[:, None]
    out_ref[...] = msg


def _nemb_kernel(x_ref, W_ref, b_ref, out_ref):
    out_ref[...] = (jnp.dot(x_ref[...], W_ref[...],
                            preferred_element_type=jnp.float32) + b_ref[...])


def _combine_kernel(upd_ref, sout_ref, Wm_ref, bm_ref, out_ref, stats_ref):
    u = upd_ref[...] * sout_ref[...]
    o = jnp.dot(u, Wm_ref[...], preferred_element_type=jnp.float32) + bm_ref[...]
    out_ref[...] = o

    @pl.when(pl.program_id(0) == 0)
    def _():
        stats_ref[...] = jnp.zeros_like(stats_ref)

    s = jnp.sum(o, axis=0, keepdims=True)
    s2 = jnp.sum(o * o, axis=0, keepdims=True)
    stats_ref[...] += jnp.concatenate([s, s2], axis=0)


def _bn_kernel(o_ref, scale_ref, shift_ref, out_ref):
    o = o_ref[...] * scale_ref[...] + shift_ref[...]
    out_ref[...] = o * jax.nn.sigmoid(o)


_ROWS = _E // 128
_RPW = 79
_ROWS_PAD = 32 * _RPW
_STR = 625


def _agg_body(msga_hbm, nemb_hbm, oidx_hbm, iidx_hbm, out_hbm,
              oid_v, iid_v, row_l, row_g, zb, acc, sem1, sem2):
    cid = lax.axis_index("c")
    sid = lax.axis_index("s")
    wid = sid * 2 + cid

    zv = jnp.zeros((16,), jnp.float32)

    def zrow(i, c):
        for j in range(8):
            zb[i, 16 * j:16 * (j + 1)] = zv
        return c

    lax.fori_loop(0, 25, zrow, 0)

    def zstripe(k, c):
        pltpu.sync_copy(zb, acc.at[pl.ds(sid * _STR + k * 25, 25)])
        return c

    lax.fori_loop(0, 25, zstripe, 0)
    plsc.subcore_barrier()

    def row(t, c):
        r = wid * _RPW + t

        @pl.when(r < _ROWS)
        def _():
            pltpu.sync_copy(oidx_hbm.at[pl.ds(r, 1)], oid_v)
            pltpu.sync_copy(iidx_hbm.at[pl.ds(r, 1)], iid_v)
            cp1 = pltpu.async_copy(nemb_hbm.at[iid_v.at[0]], row_g, sem1)
            cp2 = pltpu.async_copy(msga_hbm.at[pl.ds(r * 128, 128)], row_l, sem2)
            cp1.wait()
            cp2.wait()
            pltpu.sync_copy(row_g, acc.at[oid_v.at[0]], add=True)
            pltpu.sync_copy(row_l, acc.at[oid_v.at[0]], add=True)

        return c

    lax.fori_loop(0, _RPW, row, 0)
    plsc.subcore_barrier()
    pltpu.sync_copy(acc.at[pl.ds(sid * _STR, _STR)],
                    out_hbm.at[cid, pl.ds(sid * _STR, _STR)])


def _aggregate(msgA, nemb2, oidx_pad, iidx_pad, interpret=False):
    return pl.kernel(
        _agg_body,
        out_type=jax.ShapeDtypeStruct((2, _N, _D), jnp.float32),
        mesh=plsc.VectorSubcoreMesh(core_axis_name="c", subcore_axis_name="s"),
        scratch_types=[
            pltpu.VMEM((1, 128), jnp.int32),
            pltpu.VMEM((1, 128), jnp.int32),
            pltpu.VMEM((128, _D), jnp.float32),
            pltpu.VMEM((128, _D), jnp.float32),
            pltpu.VMEM((25, _D), jnp.float32),
            pltpu.VMEM_SHARED((_N, _D), jnp.float32),
            pltpu.SemaphoreType.DMA,
            pltpu.SemaphoreType.DMA,
        ],
    )(msgA, nemb2, oidx_pad, iidx_pad)


def _scatter_min_arg(vals, seg, num_segments):
    minv = jax.ops.segment_min(vals, seg, num_segments=num_segments)
    cand = jnp.where(vals <= minv[seg], jnp.arange(_E), _E)
    arg = jax.ops.segment_min(cand, seg, num_segments=num_segments)
    return jnp.where(arg >= _E, 0, arg)


def kernel(x, node_pos, W_in, b_in, Wl1, bl1, Wl2, bl2, Wg1, bg1, Wg2, bg2,
           Wd, bd, Wm, bm, gamma, beta, edge_index):
    node_in = edge_index[0]
    node_out = edge_index[1]

    Wloc = Wl2 @ Wd[:_D]
    Wglb = Wg2 @ Wd[_D:]
    beff = (bl2 @ Wd[:_D] + bg2 @ Wd[_D:] + bd)[None, :]

    vectors = node_pos[node_out] - node_pos[node_in]
    distance = jnp.sqrt(jnp.sum(vectors * vectors, axis=-1) + 1e-12)

    source_index_f = _scatter_min_arg(distance, node_out, _N)
    punish_s = jnp.zeros_like(distance).at[source_index_f].set(_CUT)
    source_index_s = _scatter_min_arg(distance + punish_s, node_out, _N)
    target_index_f = _scatter_min_arg(distance, node_in, _N)
    punish_t = jnp.zeros_like(distance).at[target_index_f].set(_CUT)
    target_index_s = _scatter_min_arg(distance + punish_t, node_in, _N)

    source_node_f = node_in[source_index_f]
    target_node_f = node_out[target_index_f]

    mask_iref = source_node_f[node_out] == node_in
    idx_iref = jnp.where(mask_iref, source_index_s[node_out],
                         source_index_f[node_out])
    mask_jref = target_node_f[node_in] == node_out
    idx_jref = jnp.where(mask_jref, target_index_s[node_in],
                         target_index_f[node_in])

    ones = jnp.ones((_E,), jnp.float32)
    deg_in = jax.ops.segment_sum(ones, node_in, num_segments=_N) + 1.0
    deg_out = jax.ops.segment_sum(ones, node_out, num_segments=_N) + 1.0
    s_in_node = 1.0 / (jnp.sqrt(deg_in) + _EPS)
    s_out = (1.0 / (jnp.sqrt(deg_out) + _EPS))[:, None]

    geo = jnp.concatenate([
        vectors.T,
        vectors[source_index_f[node_out]].T,
        vectors[idx_iref].T,
        vectors[idx_jref].T,
        distance[None, :],
        s_in_node[node_in][None, :],
    ], axis=0)

    node_emb = pl.pallas_call(
        _nemb_kernel,
        grid=(_N // _NBLK,),
        in_specs=[pl.BlockSpec((_NBLK, _D), lambda i: (i, 0)),
                  pl.BlockSpec((_D, _D), lambda i: (0, 0)),
                  pl.BlockSpec((1, _D), lambda i: (0, 0))],
        out_specs=pl.BlockSpec((_NBLK, _D), lambda i: (i, 0)),
        out_shape=jax.ShapeDtypeStruct((_N, _D), jnp.float32),
    )(x, W_in, b_in[None, :])

    nemb_e = node_emb[node_in]

    msg = pl.pallas_call(
        _edge_kernel,
        grid=(_E // _EBLK,),
        in_specs=[pl.BlockSpec((14, _EBLK), lambda i: (0, i)),
                  pl.BlockSpec((_EBLK, _D), lambda i: (i, 0)),
                  pl.BlockSpec((12, _MID), lambda i: (0, 0)),
                  pl.BlockSpec((6, _MID), lambda i: (0, 0)),
                  pl.BlockSpec((_MID, _D), lambda i: (0, 0)),
                  pl.BlockSpec((_MID, _D), lambda i: (0, 0)),
                  pl.BlockSpec((1, _D), lambda i: (0, 0))],
        out_specs=pl.BlockSpec((_EBLK, _D), lambda i: (i, 0)),
        out_shape=jax.ShapeDtypeStruct((_E, _D), jnp.float32),
    )(geo, nemb_e, Wl1, Wg1, Wloc, Wglb, beff)

    update = jax.ops.segment_sum(msg, node_out, num_segments=_N)

    out1, stats = pl.pallas_call(
        _combine_kernel,
        grid=(_N // _NBLK,),
        in_specs=[pl.BlockSpec((_NBLK, _D), lambda i: (i, 0)),
                  pl.BlockSpec((_NBLK, 1), lambda i: (i, 0)),
                  pl.BlockSpec((_D, _D), lambda i: (0, 0)),
                  pl.BlockSpec((1, _D), lambda i: (0, 0))],
        out_specs=[pl.BlockSpec((_NBLK, _D), lambda i: (i, 0)),
                   pl.BlockSpec((2, _D), lambda i: (0, 0))],
        out_shape=[jax.ShapeDtypeStruct((_N, _D), jnp.float32),
                   jax.ShapeDtypeStruct((2, _D), jnp.float32)],
    )(update, s_out, Wm, bm[None, :])

    mean = stats[0] / _N
    var = stats[1] / _N - mean * mean
    scale = gamma / jnp.sqrt(var + 1e-5)
    shift = beta - mean * scale

    out = pl.pallas_call(
        _bn_kernel,
        grid=(_N // _NBLK,),
        in_specs=[pl.BlockSpec((_NBLK, _D), lambda i: (i, 0)),
                  pl.BlockSpec((1, _D), lambda i: (0, 0)),
                  pl.BlockSpec((1, _D), lambda i: (0, 0))],
        out_specs=pl.BlockSpec((_NBLK, _D), lambda i: (i, 0)),
        out_shape=jax.ShapeDtypeStruct((_N, _D), jnp.float32),
    )(out1, scale[None, :], shift[None, :])
    return out


# speedup vs baseline: 14.8280x; 14.7463x over previous
import functools
from math import pi as PI

import jax
import jax.numpy as jnp
import numpy as np
from jax import lax
from jax.experimental import pallas as pl
from jax.experimental.pallas import tpu as pltpu
from jax.experimental.pallas import tpu_sc as plsc

_N = 10000
_E = 320000
_D = 128
_MID = 64
_CUT = 8.0
_EPS = 1e-10

_EBLK = 2560
_NBLK = 1000

_ROWS = _E // 128
_RPW = 79
_ROWS_PAD = 32 * _RPW
_NPAD = 10240
_STR = 640
_BIGF = 1e30
_BIGE = 1 << 30

_C0 = 0.5 / np.sqrt(PI)
_C1 = np.sqrt(3.0 / (4.0 * PI))


def _geoA_body(oidx, iidx, px_h, py_h, pz_h, ss_out, deg_out,
               pxv, pyv, pzv, krow, irow, ssrow, ones_v, zb,
               din_sp, dout_sp):
    cid = lax.axis_index("c")
    sid = lax.axis_index("s")
    wid = sid * 2 + cid

    pltpu.sync_copy(px_h, pxv)
    pltpu.sync_copy(py_h, pyv)
    pltpu.sync_copy(pz_h, pzv)

    ov = jnp.full((16,), 1.0, jnp.float32)
    for j in range(8):
        ones_v[0, 16 * j:16 * (j + 1)] = ov
    zv = jnp.zeros((16,), jnp.float32)

    def zrow(i, c):
        zb[pl.ds(i * 16, 16)] = zv
        return c

    lax.fori_loop(0, _STR // 16, zrow, 0)
    pltpu.sync_copy(zb, din_sp.at[pl.ds(sid * _STR, _STR)])
    pltpu.sync_copy(zb, dout_sp.at[pl.ds(sid * _STR, _STR)])
    plsc.subcore_barrier()

    def row(t, c):
        r = wid * _RPW + t

        @pl.when(r < _ROWS)
        def _():
            pltpu.sync_copy(oidx.at[r], krow)
            pltpu.sync_copy(iidx.at[r], irow)
            for j in range(8):
                s = pl.ds(16 * j, 16)
                o16 = krow[0, s]
                i16 = irow[0, s]
                vx = plsc.load_gather(pxv, [o16]) - plsc.load_gather(pxv, [i16])
                vy = plsc.load_gather(pyv, [o16]) - plsc.load_gather(pyv, [i16])
                vz = plsc.load_gather(pzv, [o16]) - plsc.load_gather(pzv, [i16])
                ssrow[0, s] = vx * vx + vy * vy + vz * vz + 1e-12
            pltpu.sync_copy(ssrow, ss_out.at[r])
            pltpu.sync_copy(ones_v.at[0], din_sp.at[irow.at[0]], add=True)
            pltpu.sync_copy(ones_v.at[0], dout_sp.at[krow.at[0]], add=True)

        return c

    lax.fori_loop(0, _RPW, row, 0)
    plsc.subcore_barrier()
    st = pl.ds(sid * _STR, _STR)
    pltpu.sync_copy(din_sp.at[st], deg_out.at[cid, 0, 0, st])
    pltpu.sync_copy(dout_sp.at[st], deg_out.at[cid, 1, 0, st])


def _stage_a(oidx3, iidx3, px, py, pz):
    return pl.kernel(
        _geoA_body,
        out_type=[jax.ShapeDtypeStruct((_ROWS, 1, 128), jnp.float32),
                  jax.ShapeDtypeStruct((2, 2, 1, _NPAD), jnp.float32)],
        mesh=plsc.VectorSubcoreMesh(core_axis_name="c", subcore_axis_name="s"),
        compiler_params=pltpu.CompilerParams(needs_layout_passes=False),
        scratch_types=[
            pltpu.VMEM((_NPAD,), jnp.float32),
            pltpu.VMEM((_NPAD,), jnp.float32),
            pltpu.VMEM((_NPAD,), jnp.float32),
            pltpu.VMEM((1, 128), jnp.int32),
            pltpu.VMEM((1, 128), jnp.int32),
            pltpu.VMEM((1, 128), jnp.float32),
            pltpu.VMEM((1, 128), jnp.float32),
            pltpu.VMEM((_STR,), jnp.float32),
            pltpu.VMEM_SHARED((_NPAD,), jnp.float32),
            pltpu.VMEM_SHARED((_NPAD,), jnp.float32),
        ],
    )(oidx3, iidx3, px, py, pz)


def _sel_body(kidx, pidx, d3, fout, iout,
              krow, prow, drow, d1t, e1t, n1t, d2t, e2t, n2t, wbuf):
    cid = lax.axis_index("c")
    sid = lax.axis_index("s")
    wid = sid * 2 + cid
    iota = lax.iota(jnp.int32, 16)
    bigf = jnp.full((16,), _BIGF, jnp.float32)
    bige = jnp.full((16,), _BIGE, jnp.int32)
    zi = jnp.zeros((16,), jnp.int32)

    def initrow(i, c):
        s = pl.ds(i * 16, 16)
        d1t[s] = bigf
        d2t[s] = bigf
        e1t[s] = bige
        e2t[s] = bige
        n1t[s] = zi
        n2t[s] = zi
        return c

    lax.fori_loop(0, _NPAD // 16, initrow, 0)

    def row(t, c):
        r = wid * _RPW + t

        @pl.when(r < _ROWS)
        def _():
            pltpu.sync_copy(kidx.at[r], krow)
            pltpu.sync_copy(pidx.at[r], prow)
            pltpu.sync_copy(d3.at[r], drow)
            for j in range(8):
                s = pl.ds(16 * j, 16)
                key = krow[0, s]
                pay = prow[0, s]
                dv = drow[0, s]
                ev = iota + (r * 128 + 16 * j)

                def cond(p):
                    return jnp.any(p != 0)

                def body(p):
                    pm = p != 0
                    plsc.store_scatter(wbuf, [key], iota, mask=pm)
                    win = plsc.load_gather(wbuf, [key]) == iota
                    m = jnp.logical_and(win, pm)
                    cd1 = plsc.load_gather(d1t, [key])
                    ce1 = plsc.load_gather(e1t, [key])
                    cn1 = plsc.load_gather(n1t, [key])
                    cd2 = plsc.load_gather(d2t, [key])
                    ce2 = plsc.load_gather(e2t, [key])
                    cn2 = plsc.load_gather(n2t, [key])
                    b1 = (dv < cd1) | ((dv == cd1) & (ev < ce1))
                    nd1 = jnp.where(b1, dv, cd1)
                    ne1 = jnp.where(b1, ev, ce1)
                    nn1 = jnp.where(b1, pay, cn1)
                    xd = jnp.where(b1, cd1, dv)
                    xe = jnp.where(b1, ce1, ev)
                    xn = jnp.where(b1, cn1, pay)
                    b2 = (xd < cd2) | ((xd == cd2) & (xe < ce2))
                    nd2 = jnp.where(b2, xd, cd2)
                    ne2 = jnp.where(b2, xe, ce2)
                    nn2 = jnp.where(b2, xn, cn2)
                    plsc.store_scatter(d1t, [key], nd1, mask=m)
                    plsc.store_scatter(e1t, [key], ne1, mask=m)
                    plsc.store_scatter(n1t, [key], nn1, mask=m)
                    plsc.store_scatter(d2t, [key], nd2, mask=m)
                    plsc.store_scatter(e2t, [key], ne2, mask=m)
                    plsc.store_scatter(n2t, [key], nn2, mask=m)
                    return jnp.where(m, 0, p)

                lax.while_loop(cond, body, jnp.ones((16,), jnp.int32))

        return c

    lax.fori_loop(0, _RPW, row, 0)
    pltpu.sync_copy(d1t, fout.at[wid, 0, 0])
    pltpu.sync_copy(d2t, fout.at[wid, 1, 0])
    pltpu.sync_copy(e1t, iout.at[wid, 0, 0])
    pltpu.sync_copy(n1t, iout.at[wid, 1, 0])
    pltpu.sync_copy(e2t, iout.at[wid, 2, 0])
    pltpu.sync_copy(n2t, iout.at[wid, 3, 0])


def _stage_c(kidx3, pidx3, d3):
    return pl.kernel(
        _sel_body,
        out_type=[jax.ShapeDtypeStruct((32, 2, 1, _NPAD), jnp.float32),
                  jax.ShapeDtypeStruct((32, 4, 1, _NPAD), jnp.int32)],
        mesh=plsc.VectorSubcoreMesh(core_axis_name="c", subcore_axis_name="s"),
        compiler_params=pltpu.CompilerParams(needs_layout_passes=False),
        scratch_types=[
            pltpu.VMEM((1, 128), jnp.int32),
            pltpu.VMEM((1, 128), jnp.int32),
            pltpu.VMEM((1, 128), jnp.float32),
            pltpu.VMEM((_NPAD,), jnp.float32),
            pltpu.VMEM((_NPAD,), jnp.int32),
            pltpu.VMEM((_NPAD,), jnp.int32),
            pltpu.VMEM((_NPAD,), jnp.float32),
            pltpu.VMEM((_NPAD,), jnp.int32),
            pltpu.VMEM((_NPAD,), jnp.int32),
            pltpu.VMEM((_NPAD,), jnp.int32),
        ],
    )(kidx3, pidx3, d3)


def _merge_kernel(fo_ref, io_ref, n0_ref, f_ref, s_ref):
    d1 = fo_ref[0, 0, 0]
    d2 = fo_ref[0, 1, 0]
    e1 = io_ref[0, 0, 0]
    n1 = io_ref[0, 1, 0]
    e2 = io_ref[0, 2, 0]
    n2 = io_ref[0, 3, 0]
    for t in range(1, 32):
        td1 = fo_ref[t, 0, 0]
        td2 = fo_ref[t, 1, 0]
        te1 = io_ref[t, 0, 0]
        tn1 = io_ref[t, 1, 0]
        te2 = io_ref[t, 2, 0]
        tn2 = io_ref[t, 3, 0]
        a = (d1 < td1) | ((d1 == td1) & (e1 <= te1))
        sd = jnp.where(a, td1, d1)
        se = jnp.where(a, te1, e1)
        sn = jnp.where(a, tn1, n1)
        rd = jnp.where(a, d2, td2)
        re = jnp.where(a, e2, te2)
        rn = jnp.where(a, n2, tn2)
        nd1 = jnp.where(a, d1, td1)
        ne1 = jnp.where(a, e1, te1)
        nn1 = jnp.where(a, n1, tn1)
        b = (sd < rd) | ((sd == rd) & (se < re))
        d2 = jnp.where(b, sd, rd)
        e2 = jnp.where(b, se, re)
        n2 = jnp.where(b, sn, rn)
        d1, e1, n1 = nd1, ne1, nn1
    empty = e1 >= _BIGE
    second = (d1 + _CUT < d2) | (((d1 + _CUT) == d2) & (e1 < e2))
    n0 = n0_ref[0, 0]
    f_node = jnp.where(empty, n0, n1)
    s_node = jnp.where(empty, n0, jnp.where(second, n1, n2))
    f_ref[...] = f_node[None, :]
    s_ref[...] = s_node[None, :]


def _stage_c2(fo, io, n0):
    ch = 2048
    return pl.pallas_call(
        _merge_kernel,
        grid=(_NPAD // ch,),
        in_specs=[pl.BlockSpec((32, 2, 1, ch), lambda i: (0, 0, 0, i)),
                  pl.BlockSpec((32, 4, 1, ch), lambda i: (0, 0, 0, i)),
                  pl.BlockSpec((1, 1), lambda i: (0, 0))],
        out_specs=[pl.BlockSpec((1, ch), lambda i: (0, i)),
                   pl.BlockSpec((1, ch), lambda i: (0, i))],
        out_shape=[jax.ShapeDtypeStruct((1, _NPAD), jnp.int32),
                   jax.ShapeDtypeStruct((1, _NPAD), jnp.int32)],
    )(fo, io, n0)


def _geoD_body(oidx, iidx, d3, px_h, py_h, pz_h, snf_h, sns_h, tnf_h, tns_h,
               deg_h, geo_out,
               pxv, pyv, pzv, snfv, snsv, tnfv, tnsv, degv,
               krow, irow, drow, stg):
    cid = lax.axis_index("c")
    sid = lax.axis_index("s")
    wid = sid * 2 + cid

    pltpu.sync_copy(px_h, pxv)
    pltpu.sync_copy(py_h, pyv)
    pltpu.sync_copy(pz_h, pzv)
    pltpu.sync_copy(snf_h, snfv)
    pltpu.sync_copy(sns_h, snsv)
    pltpu.sync_copy(tnf_h, tnfv)
    pltpu.sync_copy(tns_h, tnsv)
    pltpu.sync_copy(deg_h, degv)

    def row(t, c):
        r = wid * _RPW + t

        @pl.when(r < _ROWS)
        def _():
            pltpu.sync_copy(oidx.at[r], krow)
            pltpu.sync_copy(iidx.at[r], irow)
            pltpu.sync_copy(d3.at[r], drow)
            for j in range(8):
                s = pl.ds(16 * j, 16)
                o16 = krow[0, s]
                i16 = irow[0, s]
                pxo = plsc.load_gather(pxv, [o16])
                pyo = plsc.load_gather(pyv, [o16])
                pzo = plsc.load_gather(pzv, [o16])
                pxi = plsc.load_gather(pxv, [i16])
                pyi = plsc.load_gather(pyv, [i16])
                pzi = plsc.load_gather(pzv, [i16])
                snf = plsc.load_gather(snfv, [o16])
                sns = plsc.load_gather(snsv, [o16])
                tnf = plsc.load_gather(tnfv, [i16])
                tns = plsc.load_gather(tnsv, [i16])
                dgi = plsc.load_gather(degv, [i16])
                irn = jnp.where(snf == i16, sns, snf)
                jrn = jnp.where(tnf == o16, tns, tnf)
                stg[0, s] = pxo - pxi
                stg[1, s] = pyo - pyi
                stg[2, s] = pzo - pzi
                stg[3, s] = pxo - plsc.load_gather(pxv, [snf])
                stg[4, s] = pyo - plsc.load_gather(pyv, [snf])
                stg[5, s] = pzo - plsc.load_gather(pzv, [snf])
                stg[6, s] = pxo - plsc.load_gather(pxv, [irn])
                stg[7, s] = pyo - plsc.load_gather(pyv, [irn])
                stg[8, s] = pzo - plsc.load_gather(pzv, [irn])
                stg[9, s] = plsc.load_gather(pxv, [jrn]) - pxi
                stg[10, s] = plsc.load_gather(pyv, [jrn]) - pyi
                stg[11, s] = plsc.load_gather(pzv, [jrn]) - pzi
                stg[12, s] = drow[0, s]
                stg[13, s] = dgi
            pltpu.sync_copy(stg, geo_out.at[pl.ds(0, 14), pl.ds(r * 128, 128)])

        return c

    lax.fori_loop(0, _RPW, row, 0)


def _stage_d(oidx3, iidx3, d3, px, py, pz, snf, sns, tnf, tns, deg_in):
    return pl.kernel(
        _geoD_body,
        out_type=jax.ShapeDtypeStruct((14, _E), jnp.float32),
        mesh=plsc.VectorSubcoreMesh(core_axis_name="c", subcore_axis_name="s"),
        compiler_params=pltpu.CompilerParams(needs_layout_passes=False),
        scratch_types=[
            pltpu.VMEM((_NPAD,), jnp.float32),
            pltpu.VMEM((_NPAD,), jnp.float32),
            pltpu.VMEM((_NPAD,), jnp.float32),
            pltpu.VMEM((_NPAD,), jnp.int32),
            pltpu.VMEM((_NPAD,), jnp.int32),
            pltpu.VMEM((_NPAD,), jnp.int32),
            pltpu.VMEM((_NPAD,), jnp.int32),
            pltpu.VMEM((_NPAD,), jnp.float32),
            pltpu.VMEM((1, 128), jnp.int32),
            pltpu.VMEM((1, 128), jnp.int32),
            pltpu.VMEM((1, 128), jnp.float32),
            pltpu.VMEM((14, 128), jnp.float32),
        ],
    )(oidx3, iidx3, d3, px, py, pz, snf, sns, tnf, tns, deg_in)


def _agg_body(msga_hbm, nemb_hbm, oidx_hbm, iidx_hbm, out_hbm,
              oid_v, iid_v, row_l, row_g, zb, acc, sem1, sem2):
    cid = lax.axis_index("c")
    sid = lax.axis_index("s")
    wid = sid * 2 + cid

    zv = jnp.zeros((16,), jnp.float32)

    def zrow(i, c):
        for j in range(8):
            zb[i, 16 * j:16 * (j + 1)] = zv
        return c

    lax.fori_loop(0, 32, zrow, 0)

    def zstripe(k, c):
        pltpu.sync_copy(zb, acc.at[pl.ds(sid * _STR + k * 32, 32)])
        return c

    lax.fori_loop(0, 20, zstripe, 0)
    plsc.subcore_barrier()

    def row(t, c):
        r = wid * _RPW + t

        @pl.when(r < _ROWS)
        def _():
            pltpu.sync_copy(oidx_hbm.at[r], oid_v)
            pltpu.sync_copy(iidx_hbm.at[r], iid_v)
            cp1 = pltpu.async_copy(nemb_hbm.at[iid_v.at[0]], row_g, sem1)
            cp2 = pltpu.async_copy(msga_hbm.at[pl.ds(r * 128, 128)], row_l, sem2)
            cp1.wait()
            cp2.wait()
            pltpu.sync_copy(row_g, acc.at[oid_v.at[0]], add=True)
            pltpu.sync_copy(row_l, acc.at[oid_v.at[0]], add=True)

        return c

    lax.fori_loop(0, _RPW, row, 0)
    plsc.subcore_barrier()
    pltpu.sync_copy(acc.at[pl.ds(sid * _STR, _STR)],
                    out_hbm.at[cid, pl.ds(sid * _STR, _STR)])


def _aggregate(msgA, nemb2, oidx3, iidx3):
    return pl.kernel(
        _agg_body,
        out_type=jax.ShapeDtypeStruct((2, _NPAD, _D), jnp.float32),
        mesh=plsc.VectorSubcoreMesh(core_axis_name="c", subcore_axis_name="s"),
        compiler_params=pltpu.CompilerParams(needs_layout_passes=False),
        scratch_types=[
            pltpu.VMEM((1, 128), jnp.int32),
            pltpu.VMEM((1, 128), jnp.int32),
            pltpu.VMEM((128, _D), jnp.float32),
            pltpu.VMEM((128, _D), jnp.float32),
            pltpu.VMEM((32, _D), jnp.float32),
            pltpu.VMEM_SHARED((_NPAD, _D), jnp.float32),
            pltpu.SemaphoreType.DMA,
            pltpu.SemaphoreType.DMA,
        ],
    )(msgA, nemb2, oidx3, iidx3)


def _dot3(a, b):
    return a[0] * b[0] + a[1] * b[1] + a[2] * b[2]


def _cross3(a, b):
    return (a[1] * b[2] - a[2] * b[1],
            a[2] * b[0] - a[0] * b[2],
            a[0] * b[1] - a[1] * b[0])


def _safe_atan2(b, a):
    guard = (a * a + b * b) < 1e-18
    a = jnp.where(guard, 1.0, a)
    b = jnp.where(guard, 0.0, b)
    return jnp.arctan2(b, a)


def _sqrt_kernel(ss_ref, d_ref):
    d_ref[...] = jnp.sqrt(ss_ref[...])


def _deg_kernel(dp_ref, din_ref, sout_ref):
    p = dp_ref[...]
    din_ref[...] = p[0, 0] + p[1, 0] + 1.0
    dout = p[0, 1] + p[1, 1] + 1.0
    sout_ref[...] = 1.0 / (jnp.sqrt(dout) + _EPS)


def _edge_kernel(geo_ref, Wl1_ref, Wg1_ref, Wloc_ref, Wglb_ref,
                 beff_ref, out_ref):
    g = geo_ref[...]
    v = (g[0], g[1], g[2])
    f = (g[3], g[4], g[5])
    r = (g[6], g[7], g[8])
    jr = (g[9], g[10], g[11])
    d = g[12]
    s_in = 1.0 / (jnp.sqrt(g[13]) + _EPS)
    nv = (-v[0], -v[1], -v[2])

    cvf = _cross3(nv, f)
    a_t = _dot3(nv, f)
    b_t = jnp.sqrt(_dot3(cvf, cvf) + 1e-12)
    theta = _safe_atan2(b_t, a_t)

    p2 = _cross3(nv, r)
    a_p = _dot3(cvf, p2)
    b_p = _dot3(_cross3(cvf, p2), v) / d
    phi = _safe_atan2(b_p, a_p)
    phi = jnp.where(phi < 0, phi + PI, phi)

    q1 = _cross3(v, jr)
    q2 = _cross3(v, r)
    a_u = _dot3(q1, q2)
    b_u = _dot3(_cross3(q1, q2), v) / d
    tau = _safe_atan2(b_u, a_u)
    tau = jnp.where(tau < 0, tau + 2.0 * PI, tau)

    pre = np.sqrt(2.0 / _CUT).astype(np.float32)
    inv_d = pre / d
    rb = [jnp.sin((n * PI / _CUT) * d) * inv_d for n in (1.0, 2.0, 3.0)]
    st, ct = jnp.sin(theta), jnp.cos(theta)
    sp, cp = jnp.sin(phi), jnp.cos(phi)
    y = [jnp.full_like(theta, _C0), _C1 * st * sp, _C1 * ct, _C1 * st * cp]
    sph = [jnp.ones_like(tau), jnp.cos(tau)]

    hl = jnp.zeros((g.shape[1], _MID), jnp.float32)
    for n in range(3):
        for m in range(4):
            feat = rb[n] * y[m]
            hl = hl + feat[:, None] * Wl1_ref[n * 4 + m, :][None, :]
    hl = hl * jax.nn.sigmoid(hl)

    hg = jnp.zeros((g.shape[1], _MID), jnp.float32)
    for n in range(3):
        for m in range(2):
            feat = rb[n] * sph[m]
            hg = hg + feat[:, None] * Wg1_ref[n * 2 + m, :][None, :]
    hg = hg * jax.nn.sigmoid(hg)

    edge_feat = (jnp.dot(hl, Wloc_ref[...], preferred_element_type=jnp.float32)
                 + jnp.dot(hg, Wglb_ref[...], preferred_element_type=jnp.float32)
                 + beff_ref[...])
    msg = (edge_feat + d[:, None]) * s_in[:, None]
    out_ref[...] = msg


def _nemb_kernel(x_ref, W_ref, b_ref, din_ref, out_ref):
    s = 1.0 / (jnp.sqrt(din_ref[...]) + _EPS)
    out_ref[...] = (jnp.dot(x_ref[...], W_ref[...],
                            preferred_element_type=jnp.float32)
                    + b_ref[...]) * s


def _combine_kernel(p_ref, sout_ref, Wm_ref, bm_ref, out_ref, stats_ref):
    u = (p_ref[0] + p_ref[1]) * sout_ref[...]
    o = jnp.dot(u, Wm_ref[...], preferred_element_type=jnp.float32) + bm_ref[...]
    out_ref[...] = o

    @pl.when(pl.program_id(0) == 0)
    def _():
        stats_ref[...] = jnp.zeros_like(stats_ref)

    s = jnp.sum(o, axis=0, keepdims=True)
    s2 = jnp.sum(o * o, axis=0, keepdims=True)
    stats_ref[...] += jnp.concatenate([s, s2], axis=0)


def _bn_kernel(o_ref, scale_ref, shift_ref, out_ref):
    o = o_ref[...] * scale_ref[...] + shift_ref[...]
    out_ref[...] = o * jax.nn.sigmoid(o)


def kernel(x, node_pos, W_in, b_in, Wl1, bl1, Wl2, bl2, Wg1, bg1, Wg2, bg2,
           Wd, bd, Wm, bm, gamma, beta, edge_index):
    node_in = edge_index[0]
    node_out = edge_index[1]

    Wloc = Wl2 @ Wd[:_D]
    Wglb = Wg2 @ Wd[_D:]
    beff = (bl2 @ Wd[:_D] + bg2 @ Wd[_D:] + bd)[None, :]

    pad = jnp.zeros((_ROWS_PAD - _ROWS, 1, 128), jnp.int32)
    oidx3 = jnp.concatenate([node_out.reshape(_ROWS, 1, 128), pad], axis=0)
    iidx3 = jnp.concatenate([node_in.reshape(_ROWS, 1, 128), pad], axis=0)
    posp = jnp.pad(node_pos, ((0, _NPAD - _N), (0, 0))).T
    px, py, pz = posp[0], posp[1], posp[2]

    ss2, degparts = _stage_a(oidx3, iidx3, px, py, pz)

    d2d = pl.pallas_call(
        _sqrt_kernel,
        grid=(_ROWS // 500,),
        in_specs=[pl.BlockSpec((500, 1, 128), lambda i: (i, 0, 0))],
        out_specs=pl.BlockSpec((500, 1, 128), lambda i: (i, 0, 0)),
        out_shape=jax.ShapeDtypeStruct((_ROWS, 1, 128), jnp.float32),
    )(ss2)

    din2, sout2 = pl.pallas_call(
        _deg_kernel,
        grid=(1,),
        in_specs=[pl.BlockSpec((2, 2, 1, _NPAD), lambda i: (0, 0, 0, 0))],
        out_specs=[pl.BlockSpec((1, _NPAD), lambda i: (0, 0)),
                   pl.BlockSpec((1, _NPAD), lambda i: (0, 0))],
        out_shape=[jax.ShapeDtypeStruct((1, _NPAD), jnp.float32),
                   jax.ShapeDtypeStruct((1, _NPAD), jnp.float32)],
    )(degparts)
    din = din2.reshape(_NPAD)
    s_out = sout2.reshape(_NPAD, 1)

    fo_s, io_s = _stage_c(oidx3, iidx3, d2d)
    fo_t, io_t = _stage_c(iidx3, oidx3, d2d)
    ni0 = node_in[:1].reshape(1, 1)
    no0 = node_out[:1].reshape(1, 1)
    snf2, sns2 = _stage_c2(fo_s, io_s, ni0)
    tnf2, tns2 = _stage_c2(fo_t, io_t, no0)

    geo = _stage_d(oidx3, iidx3, d2d, px, py, pz,
                   snf2.reshape(_NPAD), sns2.reshape(_NPAD),
                   tnf2.reshape(_NPAD), tns2.reshape(_NPAD), din)

    msgA = pl.pallas_call(
        _edge_kernel,
        grid=(_E // _EBLK,),
        in_specs=[pl.BlockSpec((14, _EBLK), lambda i: (0, i)),
                  pl.BlockSpec((12, _MID), lambda i: (0, 0)),
                  pl.BlockSpec((6, _MID), lambda i: (0, 0)),
                  pl.BlockSpec((_MID, _D), lambda i: (0, 0)),
                  pl.BlockSpec((_MID, _D), lambda i: (0, 0)),
                  pl.BlockSpec((1, _D), lambda i: (0, 0))],
        out_specs=pl.BlockSpec((_EBLK, _D), lambda i: (i, 0)),
        out_shape=jax.ShapeDtypeStruct((_E, _D), jnp.float32),
    )(geo, Wl1, Wg1, Wloc, Wglb, beff)

    nemb2 = pl.pallas_call(
        _nemb_kernel,
        grid=(_N // _NBLK,),
        in_specs=[pl.BlockSpec((_NBLK, _D), lambda i: (i, 0)),
                  pl.BlockSpec((_D, _D), lambda i: (0, 0)),
                  pl.BlockSpec((1, _D), lambda i: (0, 0)),
                  pl.BlockSpec((_NBLK, 1), lambda i: (i, 0))],
        out_specs=pl.BlockSpec((_NBLK, _D), lambda i: (i, 0)),
        out_shape=jax.ShapeDtypeStruct((_N, _D), jnp.float32),
    )(x, W_in, b_in[None, :], din[:_N].reshape(_N, 1))

    parts = _aggregate(msgA, nemb2, oidx3, iidx3)

    out1, stats = pl.pallas_call(
        _combine_kernel,
        grid=(_N // _NBLK,),
        in_specs=[pl.BlockSpec((2, _NBLK, _D), lambda i: (0, i, 0)),
                  pl.BlockSpec((_NBLK, 1), lambda i: (i, 0)),
                  pl.BlockSpec((_D, _D), lambda i: (0, 0)),
                  pl.BlockSpec((1, _D), lambda i: (0, 0))],
        out_specs=[pl.BlockSpec((_NBLK, _D), lambda i: (i, 0)),
                   pl.BlockSpec((2, _D), lambda i: (0, 0))],
        out_shape=[jax.ShapeDtypeStruct((_N, _D), jnp.float32),
                   jax.ShapeDtypeStruct((2, _D), jnp.float32)],
    )(parts, s_out, Wm, bm[None, :])

    mean = stats[0] / _N
    var = stats[1] / _N - mean * mean
    scale = gamma / jnp.sqrt(var + 1e-5)
    shift = beta - mean * scale

    out = pl.pallas_call(
        _bn_kernel,
        grid=(_N // _NBLK,),
        in_specs=[pl.BlockSpec((_NBLK, _D), lambda i: (i, 0)),
                  pl.BlockSpec((1, _D), lambda i: (0, 0)),
                  pl.BlockSpec((1, _D), lambda i: (0, 0))],
        out_specs=pl.BlockSpec((_NBLK, _D), lambda i: (i, 0)),
        out_shape=jax.ShapeDtypeStruct((_N, _D), jnp.float32),
    )(out1, scale[None, :], shift[None, :])
    return out


# speedup vs baseline: 24.7776x; 1.6710x over previous
import functools
from math import pi as PI

import jax
import jax.numpy as jnp
import numpy as np
from jax import lax
from jax.experimental import pallas as pl
from jax.experimental.pallas import tpu as pltpu
from jax.experimental.pallas import tpu_sc as plsc

_N = 10000
_E = 320000
_D = 128
_MID = 64
_CUT = 8.0
_EPS = 1e-10

_EBLK = 2560
_NBLK = 1000

_ROWS = _E // 128
_RPW = 82
_ROWS_PAD = 32 * _RPW
_NPAD = 10240
_STR = 640
_BIGF = 1e30
_BIGE = 1 << 30

_C0 = 0.5 / np.sqrt(PI)
_C1 = np.sqrt(3.0 / (4.0 * PI))


def _geoA_body(oidx, iidx, px_h, py_h, pz_h, ss_out, deg_out,
               pxv, pyv, pzv, krow, irow, ssrow, ones_v, zb,
               din_sp, dout_sp):
    cid = lax.axis_index("c")
    sid = lax.axis_index("s")
    wid = sid * 2 + cid

    pltpu.sync_copy(px_h, pxv)
    pltpu.sync_copy(py_h, pyv)
    pltpu.sync_copy(pz_h, pzv)

    ov = jnp.full((16,), 1.0, jnp.float32)
    for j in range(8):
        ones_v[0, 16 * j:16 * (j + 1)] = ov
    zv = jnp.zeros((16,), jnp.float32)

    def zrow(i, c):
        zb[pl.ds(i * 16, 16)] = zv
        return c

    lax.fori_loop(0, _STR // 16, zrow, 0)
    pltpu.sync_copy(zb, din_sp.at[pl.ds(sid * _STR, _STR)])
    pltpu.sync_copy(zb, dout_sp.at[pl.ds(sid * _STR, _STR)])
    plsc.subcore_barrier()

    def row(t, c):
        r = wid * _RPW + t

        @pl.when(r < _ROWS)
        def _():
            pltpu.sync_copy(oidx.at[r], krow)
            pltpu.sync_copy(iidx.at[r], irow)
            for j in range(8):
                s = pl.ds(16 * j, 16)
                o16 = krow[0, s]
                i16 = irow[0, s]
                vx = plsc.load_gather(pxv, [o16]) - plsc.load_gather(pxv, [i16])
                vy = plsc.load_gather(pyv, [o16]) - plsc.load_gather(pyv, [i16])
                vz = plsc.load_gather(pzv, [o16]) - plsc.load_gather(pzv, [i16])
                ssrow[0, s] = vx * vx + vy * vy + vz * vz + 1e-12
            pltpu.sync_copy(ssrow, ss_out.at[r])
            pltpu.sync_copy(ones_v.at[0], din_sp.at[irow.at[0]], add=True)
            pltpu.sync_copy(ones_v.at[0], dout_sp.at[krow.at[0]], add=True)

        return c

    lax.fori_loop(0, _RPW, row, 0)
    plsc.subcore_barrier()
    st = pl.ds(sid * _STR, _STR)
    pltpu.sync_copy(din_sp.at[st], deg_out.at[cid, 0, 0, st])
    pltpu.sync_copy(dout_sp.at[st], deg_out.at[cid, 1, 0, st])


def _stage_a(oidx3, iidx3, px, py, pz):
    return pl.kernel(
        _geoA_body,
        out_type=[jax.ShapeDtypeStruct((_ROWS, 1, 128), jnp.float32),
                  jax.ShapeDtypeStruct((2, 2, 1, _NPAD), jnp.float32)],
        mesh=plsc.VectorSubcoreMesh(core_axis_name="c", subcore_axis_name="s"),
        compiler_params=pltpu.CompilerParams(needs_layout_passes=False),
        scratch_types=[
            pltpu.VMEM((_NPAD,), jnp.float32),
            pltpu.VMEM((_NPAD,), jnp.float32),
            pltpu.VMEM((_NPAD,), jnp.float32),
            pltpu.VMEM((1, 128), jnp.int32),
            pltpu.VMEM((1, 128), jnp.int32),
            pltpu.VMEM((1, 128), jnp.float32),
            pltpu.VMEM((1, 128), jnp.float32),
            pltpu.VMEM((_STR,), jnp.float32),
            pltpu.VMEM_SHARED((_NPAD,), jnp.float32),
            pltpu.VMEM_SHARED((_NPAD,), jnp.float32),
        ],
    )(oidx3, iidx3, px, py, pz)


def _sel_body(kidx, pidx, d3, fout, iout,
              krow, prow, drow, d1t, e1t, n1t, d2t, e2t, n2t, wbuf):
    cid = lax.axis_index("c")
    sid = lax.axis_index("s")
    wid = sid * 2 + cid
    iota = lax.iota(jnp.int32, 16)
    bigf = jnp.full((16,), _BIGF, jnp.float32)
    bige = jnp.full((16,), _BIGE, jnp.int32)
    zi = jnp.zeros((16,), jnp.int32)

    def initrow(i, c):
        s = pl.ds(i * 16, 16)
        d1t[s] = bigf
        d2t[s] = bigf
        e1t[s] = bige
        e2t[s] = bige
        n1t[s] = zi
        n2t[s] = zi
        return c

    lax.fori_loop(0, _NPAD // 16, initrow, 0)

    def row(t, c):
        r = wid * _RPW + t

        @pl.when(r < _ROWS)
        def _():
            pltpu.sync_copy(kidx.at[r], krow)
            pltpu.sync_copy(pidx.at[r], prow)
            pltpu.sync_copy(d3.at[r], drow)
            for j in range(8):
                s = pl.ds(16 * j, 16)
                key = krow[0, s]
                pay = prow[0, s]
                dv = drow[0, s]
                ev = iota + (r * 128 + 16 * j)

                def cond(p):
                    return jnp.any(p != 0)

                def body(p):
                    pm = p != 0
                    plsc.store_scatter(wbuf, [key], iota, mask=pm)
                    win = plsc.load_gather(wbuf, [key]) == iota
                    m = jnp.logical_and(win, pm)
                    cd1 = plsc.load_gather(d1t, [key])
                    ce1 = plsc.load_gather(e1t, [key])
                    cn1 = plsc.load_gather(n1t, [key])
                    cd2 = plsc.load_gather(d2t, [key])
                    ce2 = plsc.load_gather(e2t, [key])
                    cn2 = plsc.load_gather(n2t, [key])
                    b1 = (dv < cd1) | ((dv == cd1) & (ev < ce1))
                    nd1 = jnp.where(b1, dv, cd1)
                    ne1 = jnp.where(b1, ev, ce1)
                    nn1 = jnp.where(b1, pay, cn1)
                    xd = jnp.where(b1, cd1, dv)
                    xe = jnp.where(b1, ce1, ev)
                    xn = jnp.where(b1, cn1, pay)
                    b2 = (xd < cd2) | ((xd == cd2) & (xe < ce2))
                    nd2 = jnp.where(b2, xd, cd2)
                    ne2 = jnp.where(b2, xe, ce2)
                    nn2 = jnp.where(b2, xn, cn2)
                    plsc.store_scatter(d1t, [key], nd1, mask=m)
                    plsc.store_scatter(e1t, [key], ne1, mask=m)
                    plsc.store_scatter(n1t, [key], nn1, mask=m)
                    plsc.store_scatter(d2t, [key], nd2, mask=m)
                    plsc.store_scatter(e2t, [key], ne2, mask=m)
                    plsc.store_scatter(n2t, [key], nn2, mask=m)
                    return jnp.where(m, 0, p)

                lax.while_loop(cond, body, jnp.ones((16,), jnp.int32))

        return c

    lax.fori_loop(0, _RPW, row, 0)
    pltpu.sync_copy(d1t, fout.at[wid, 0, 0])
    pltpu.sync_copy(d2t, fout.at[wid, 1, 0])
    pltpu.sync_copy(e1t, iout.at[wid, 0, 0])
    pltpu.sync_copy(n1t, iout.at[wid, 1, 0])
    pltpu.sync_copy(e2t, iout.at[wid, 2, 0])
    pltpu.sync_copy(n2t, iout.at[wid, 3, 0])


def _stage_c(kidx3, pidx3, d3):
    return pl.kernel(
        _sel_body,
        out_type=[jax.ShapeDtypeStruct((32, 2, 1, _NPAD), jnp.float32),
                  jax.ShapeDtypeStruct((32, 4, 1, _NPAD), jnp.int32)],
        mesh=plsc.VectorSubcoreMesh(core_axis_name="c", subcore_axis_name="s"),
        compiler_params=pltpu.CompilerParams(needs_layout_passes=False),
        scratch_types=[
            pltpu.VMEM((1, 128), jnp.int32),
            pltpu.VMEM((1, 128), jnp.int32),
            pltpu.VMEM((1, 128), jnp.float32),
            pltpu.VMEM((_NPAD,), jnp.float32),
            pltpu.VMEM((_NPAD,), jnp.int32),
            pltpu.VMEM((_NPAD,), jnp.int32),
            pltpu.VMEM((_NPAD,), jnp.float32),
            pltpu.VMEM((_NPAD,), jnp.int32),
            pltpu.VMEM((_NPAD,), jnp.int32),
            pltpu.VMEM((_NPAD,), jnp.int32),
        ],
    )(kidx3, pidx3, d3)


def _merge_kernel(fo_ref, io_ref, n0_ref, f_ref, s_ref):
    d1 = fo_ref[0, 0, 0]
    d2 = fo_ref[0, 1, 0]
    e1 = io_ref[0, 0, 0]
    n1 = io_ref[0, 1, 0]
    e2 = io_ref[0, 2, 0]
    n2 = io_ref[0, 3, 0]
    for t in range(1, 32):
        td1 = fo_ref[t, 0, 0]
        td2 = fo_ref[t, 1, 0]
        te1 = io_ref[t, 0, 0]
        tn1 = io_ref[t, 1, 0]
        te2 = io_ref[t, 2, 0]
        tn2 = io_ref[t, 3, 0]
        a = (d1 < td1) | ((d1 == td1) & (e1 <= te1))
        sd = jnp.where(a, td1, d1)
        se = jnp.where(a, te1, e1)
        sn = jnp.where(a, tn1, n1)
        rd = jnp.where(a, d2, td2)
        re = jnp.where(a, e2, te2)
        rn = jnp.where(a, n2, tn2)
        nd1 = jnp.where(a, d1, td1)
        ne1 = jnp.where(a, e1, te1)
        nn1 = jnp.where(a, n1, tn1)
        b = (sd < rd) | ((sd == rd) & (se < re))
        d2 = jnp.where(b, sd, rd)
        e2 = jnp.where(b, se, re)
        n2 = jnp.where(b, sn, rn)
        d1, e1, n1 = nd1, ne1, nn1
    empty = e1 >= _BIGE
    second = (d1 + _CUT < d2) | (((d1 + _CUT) == d2) & (e1 < e2))
    n0 = n0_ref[0, 0]
    f_node = jnp.where(empty, n0, n1)
    s_node = jnp.where(empty, n0, jnp.where(second, n1, n2))
    f_ref[...] = f_node[None, :]
    s_ref[...] = s_node[None, :]


def _stage_c2(fo, io, n0):
    ch = 2048
    return pl.pallas_call(
        _merge_kernel,
        grid=(_NPAD // ch,),
        in_specs=[pl.BlockSpec((32, 2, 1, ch), lambda i: (0, 0, 0, i)),
                  pl.BlockSpec((32, 4, 1, ch), lambda i: (0, 0, 0, i)),
                  pl.BlockSpec((1, 1), lambda i: (0, 0))],
        out_specs=[pl.BlockSpec((1, ch), lambda i: (0, i)),
                   pl.BlockSpec((1, ch), lambda i: (0, i))],
        out_shape=[jax.ShapeDtypeStruct((1, _NPAD), jnp.int32),
                   jax.ShapeDtypeStruct((1, _NPAD), jnp.int32)],
    )(fo, io, n0)


def _geoD_body(oidx, iidx, d3, px_h, py_h, pz_h, snf_h, sns_h, tnf_h, tns_h,
               deg_h, geo_out,
               pxv, pyv, pzv, snfv, snsv, tnfv, tnsv, degv,
               krow, irow, drow, stg):
    cid = lax.axis_index("c")
    sid = lax.axis_index("s")
    wid = sid * 2 + cid

    pltpu.sync_copy(px_h, pxv)
    pltpu.sync_copy(py_h, pyv)
    pltpu.sync_copy(pz_h, pzv)
    pltpu.sync_copy(snf_h, snfv)
    pltpu.sync_copy(sns_h, snsv)
    pltpu.sync_copy(tnf_h, tnfv)
    pltpu.sync_copy(tns_h, tnsv)
    pltpu.sync_copy(deg_h, degv)

    def row(t, c):
        r = wid * _RPW + t

        @pl.when(r < _ROWS)
        def _():
            pltpu.sync_copy(oidx.at[r], krow)
            pltpu.sync_copy(iidx.at[r], irow)
            pltpu.sync_copy(d3.at[r], drow)
            for j in range(8):
                s = pl.ds(16 * j, 16)
                o16 = krow[0, s]
                i16 = irow[0, s]
                pxo = plsc.load_gather(pxv, [o16])
                pyo = plsc.load_gather(pyv, [o16])
                pzo = plsc.load_gather(pzv, [o16])
                pxi = plsc.load_gather(pxv, [i16])
                pyi = plsc.load_gather(pyv, [i16])
                pzi = plsc.load_gather(pzv, [i16])
                snf = plsc.load_gather(snfv, [o16])
                sns = plsc.load_gather(snsv, [o16])
                tnf = plsc.load_gather(tnfv, [i16])
                tns = plsc.load_gather(tnsv, [i16])
                dgi = plsc.load_gather(degv, [i16])
                irn = jnp.where(snf == i16, sns, snf)
                jrn = jnp.where(tnf == o16, tns, tnf)
                stg[0, s] = pxo - pxi
                stg[1, s] = pyo - pyi
                stg[2, s] = pzo - pzi
                stg[3, s] = pxo - plsc.load_gather(pxv, [snf])
                stg[4, s] = pyo - plsc.load_gather(pyv, [snf])
                stg[5, s] = pzo - plsc.load_gather(pzv, [snf])
                stg[6, s] = pxo - plsc.load_gather(pxv, [irn])
                stg[7, s] = pyo - plsc.load_gather(pyv, [irn])
                stg[8, s] = pzo - plsc.load_gather(pzv, [irn])
                stg[9, s] = plsc.load_gather(pxv, [jrn]) - pxi
                stg[10, s] = plsc.load_gather(pyv, [jrn]) - pyi
                stg[11, s] = plsc.load_gather(pzv, [jrn]) - pzi
                stg[12, s] = drow[0, s]
                stg[13, s] = dgi
            pltpu.sync_copy(stg, geo_out.at[pl.ds(0, 14), pl.ds(r * 128, 128)])

        return c

    lax.fori_loop(0, _RPW, row, 0)


def _stage_d(oidx3, iidx3, d3, px, py, pz, snf, sns, tnf, tns, deg_in):
    return pl.kernel(
        _geoD_body,
        out_type=jax.ShapeDtypeStruct((14, _E), jnp.float32),
        mesh=plsc.VectorSubcoreMesh(core_axis_name="c", subcore_axis_name="s"),
        compiler_params=pltpu.CompilerParams(needs_layout_passes=False),
        scratch_types=[
            pltpu.VMEM((_NPAD,), jnp.float32),
            pltpu.VMEM((_NPAD,), jnp.float32),
            pltpu.VMEM((_NPAD,), jnp.float32),
            pltpu.VMEM((_NPAD,), jnp.int32),
            pltpu.VMEM((_NPAD,), jnp.int32),
            pltpu.VMEM((_NPAD,), jnp.int32),
            pltpu.VMEM((_NPAD,), jnp.int32),
            pltpu.VMEM((_NPAD,), jnp.float32),
            pltpu.VMEM((1, 128), jnp.int32),
            pltpu.VMEM((1, 128), jnp.int32),
            pltpu.VMEM((1, 128), jnp.float32),
            pltpu.VMEM((14, 128), jnp.float32),
        ],
    )(oidx3, iidx3, d3, px, py, pz, snf, sns, tnf, tns, deg_in)


def _agg_stream_body(src_hbm, oidx_hbm, iidx_hbm, out_hbm,
                     oid_v, iid_v, rowb, zb, acc, sem1, gather):
    cid = lax.axis_index("c")
    sid = lax.axis_index("s")
    wid = sid * 2 + cid

    zv = jnp.zeros((16,), jnp.float32)

    def zrow(i, c):
        for j in range(8):
            zb[i, 16 * j:16 * (j + 1)] = zv
        return c

    lax.fori_loop(0, 16, zrow, 0)

    def zstripe(k, c):
        pltpu.sync_copy(zb, acc.at[pl.ds(sid * _STR + k * 16, 16)])
        return c

    lax.fori_loop(0, 40, zstripe, 0)
    plsc.subcore_barrier()

    def group(tg, c):
        base = wid * _RPW + tg * 2
        pltpu.sync_copy(oidx_hbm.at[pl.ds(base, 2)], oid_v)
        if gather:
            pltpu.sync_copy(iidx_hbm.at[pl.ds(base, 2)], iid_v)

        for b in range(2):
            r = base + b

            @pl.when(r < _ROWS)
            def _(b=b, r=r):
                if gather:
                    pltpu.async_copy(src_hbm.at[iid_v.at[b, 0]],
                                     rowb.at[b], sem1)
                else:
                    pltpu.async_copy(src_hbm.at[pl.ds(r * 128, 128)],
                                     rowb.at[b], sem1)

        for b in range(2):
            r = base + b

            @pl.when(r < _ROWS)
            def _(b=b, r=r):
                pltpu.make_async_copy(src_hbm.at[pl.ds(0, 128)],
                                      rowb.at[b], sem1).wait()
                pltpu.sync_copy(rowb.at[b], acc.at[oid_v.at[b, 0]], add=True)

        return c

    lax.fori_loop(0, _RPW // 2, group, 0)
    plsc.subcore_barrier()
    pltpu.sync_copy(acc.at[pl.ds(sid * _STR, _STR)],
                    out_hbm.at[cid, pl.ds(sid * _STR, _STR)])


def _aggregate_stream(src, oidx3, iidx3, gather):
    body = functools.partial(_agg_stream_body, gather=gather)
    return pl.kernel(
        body,
        out_type=jax.ShapeDtypeStruct((2, _NPAD, _D), jnp.float32),
        mesh=plsc.VectorSubcoreMesh(core_axis_name="c", subcore_axis_name="s"),
        compiler_params=pltpu.CompilerParams(needs_layout_passes=False),
        scratch_types=[
            pltpu.VMEM((2, 1, 128), jnp.int32),
            pltpu.VMEM((2, 1, 128), jnp.int32),
            pltpu.VMEM((2, 128, _D), jnp.float32),
            pltpu.VMEM((16, _D), jnp.float32),
            pltpu.VMEM_SHARED((_NPAD, _D), jnp.float32),
            pltpu.SemaphoreType.DMA,
        ],
    )(src, oidx3, iidx3)


def _dot3(a, b):
    return a[0] * b[0] + a[1] * b[1] + a[2] * b[2]


def _cross3(a, b):
    return (a[1] * b[2] - a[2] * b[1],
            a[2] * b[0] - a[0] * b[2],
            a[0] * b[1] - a[1] * b[0])


def _safe_atan2(b, a):
    guard = (a * a + b * b) < 1e-18
    a = jnp.where(guard, 1.0, a)
    b = jnp.where(guard, 0.0, b)
    return jnp.arctan2(b, a)


def _mxu3(a, b, dn):
    ah16 = a.astype(jnp.bfloat16)
    ah = ah16.astype(jnp.float32)
    al16 = (a - ah).astype(jnp.bfloat16)
    bh16 = b.astype(jnp.bfloat16)
    bh = bh16.astype(jnp.float32)
    bl16 = (b - bh).astype(jnp.bfloat16)
    f32 = jnp.float32
    return (lax.dot_general(ah16, bh16, dn, preferred_element_type=f32)
            + lax.dot_general(al16, bh16, dn, preferred_element_type=f32)
            + lax.dot_general(ah16, bl16, dn, preferred_element_type=f32))


def _sqrt_kernel(ss_ref, d_ref):
    d_ref[...] = jnp.sqrt(ss_ref[...])


def _deg_kernel(dp_ref, din_ref, sout_ref):
    p = dp_ref[...]
    din_ref[...] = p[0, 0] + p[1, 0] + 1.0
    dout = p[0, 1] + p[1, 1] + 1.0
    sout_ref[...] = 1.0 / (jnp.sqrt(dout) + _EPS)


def _edge_kernel(geo_ref, Wl1_ref, Wg1_ref, Wloc_ref, Wglb_ref, out_ref):
    g = geo_ref[...]
    v = (g[0], g[1], g[2])
    f = (g[3], g[4], g[5])
    r = (g[6], g[7], g[8])
    jr = (g[9], g[10], g[11])
    d = g[12]
    s_in = 1.0 / (jnp.sqrt(g[13]) + _EPS)
    nv = (-v[0], -v[1], -v[2])

    def _sincos(b, a):
        g = (a * a + b * b) < 1e-18
        a = jnp.where(g, 1.0, a)
        b = jnp.where(g, 0.0, b)
        h = 1.0 / jnp.sqrt(a * a + b * b)
        return b * h, a * h

    cvf = _cross3(nv, f)
    a_t = _dot3(nv, f)
    b_t = jnp.sqrt(_dot3(cvf, cvf) + 1e-12)
    st, ct = _sincos(b_t, a_t)

    p2 = _cross3(nv, r)
    a_p = _dot3(cvf, p2)
    b_p = _dot3(_cross3(cvf, p2), v) / d
    sp0, cp0 = _sincos(b_p, a_p)
    neg = b_p < 0
    sp = jnp.where(neg, -sp0, sp0)
    cp = jnp.where(neg, -cp0, cp0)

    q1 = _cross3(v, jr)
    q2 = _cross3(v, r)
    a_u = _dot3(q1, q2)
    b_u = _dot3(_cross3(q1, q2), v) / d
    _, ctau = _sincos(b_u, a_u)

    pre = np.sqrt(2.0 / _CUT).astype(np.float32)
    inv_d = pre / d
    rb = [jnp.sin((n * PI / _CUT) * d) * inv_d for n in (1.0, 2.0, 3.0)]
    y = [jnp.full_like(st, _C0), _C1 * st * sp, _C1 * ct, _C1 * st * cp]
    sph = [jnp.ones_like(ctau), ctau]

    floc = jnp.stack([rb[n] * y[m] for n in range(3) for m in range(4)], axis=0)
    fglb = jnp.stack([rb[n] * sph[m] for n in range(3) for m in range(2)], axis=0)
    dn1 = (((1,), (0,)), ((), ()))
    hl = _mxu3(Wl1_ref[...], floc, dn1)
    hl = hl * jax.nn.sigmoid(hl) * s_in[None, :]
    hg = _mxu3(Wg1_ref[...], fglb, dn1)
    hg = hg * jax.nn.sigmoid(hg) * s_in[None, :]
    hg2 = jnp.concatenate([hg, (d * s_in)[None, :], s_in[None, :]], axis=0)

    dn0 = (((0,), (0,)), ((), ()))
    msg = _mxu3(hl, Wloc_ref[...], dn0) + _mxu3(hg2, Wglb_ref[...], dn0)
    out_ref[...] = msg


def _nemb_kernel(x_ref, W_ref, b_ref, din_ref, out_ref):
    s = 1.0 / (jnp.sqrt(din_ref[...]) + _EPS)
    out_ref[...] = (jnp.dot(x_ref[...], W_ref[...],
                            preferred_element_type=jnp.float32)
                    + b_ref[...]) * s


def _combine_kernel(p_ref, q_ref, sout_ref, Wm_ref, bm_ref, out_ref,
                    stats_ref):
    u = ((p_ref[0] + p_ref[1]) + (q_ref[0] + q_ref[1])) * sout_ref[...]
    o = jnp.dot(u, Wm_ref[...], preferred_element_type=jnp.float32) + bm_ref[...]
    out_ref[...] = o

    @pl.when(pl.program_id(0) == 0)
    def _():
        stats_ref[...] = jnp.zeros_like(stats_ref)

    s = jnp.sum(o, axis=0, keepdims=True)
    s2 = jnp.sum(o * o, axis=0, keepdims=True)
    stats_ref[...] += jnp.concatenate([s, s2], axis=0)


def _bn_kernel(o_ref, scale_ref, shift_ref, out_ref):
    o = o_ref[...] * scale_ref[...] + shift_ref[...]
    out_ref[...] = o * jax.nn.sigmoid(o)


def kernel(x, node_pos, W_in, b_in, Wl1, bl1, Wl2, bl2, Wg1, bg1, Wg2, bg2,
           Wd, bd, Wm, bm, gamma, beta, edge_index):
    node_in = edge_index[0]
    node_out = edge_index[1]

    Wloc = Wl2 @ Wd[:_D]
    beff = (bl2 @ Wd[:_D] + bg2 @ Wd[_D:] + bd)[None, :]
    Wglb2 = jnp.concatenate([Wg2 @ Wd[_D:], jnp.ones((1, _D), jnp.float32),
                             beff], axis=0)
    Wl1T = Wl1.T
    Wg1T = Wg1.T

    pad = jnp.zeros((_ROWS_PAD - _ROWS, 1, 128), jnp.int32)
    oidx3 = jnp.concatenate([node_out.reshape(_ROWS, 1, 128), pad], axis=0)
    iidx3 = jnp.concatenate([node_in.reshape(_ROWS, 1, 128), pad], axis=0)
    posp = jnp.pad(node_pos, ((0, _NPAD - _N), (0, 0))).T
    px, py, pz = posp[0], posp[1], posp[2]

    ss2, degparts = _stage_a(oidx3, iidx3, px, py, pz)

    d2d = pl.pallas_call(
        _sqrt_kernel,
        grid=(_ROWS // 500,),
        in_specs=[pl.BlockSpec((500, 1, 128), lambda i: (i, 0, 0))],
        out_specs=pl.BlockSpec((500, 1, 128), lambda i: (i, 0, 0)),
        out_shape=jax.ShapeDtypeStruct((_ROWS, 1, 128), jnp.float32),
    )(ss2)

    din2, sout2 = pl.pallas_call(
        _deg_kernel,
        grid=(1,),
        in_specs=[pl.BlockSpec((2, 2, 1, _NPAD), lambda i: (0, 0, 0, 0))],
        out_specs=[pl.BlockSpec((1, _NPAD), lambda i: (0, 0)),
                   pl.BlockSpec((1, _NPAD), lambda i: (0, 0))],
        out_shape=[jax.ShapeDtypeStruct((1, _NPAD), jnp.float32),
                   jax.ShapeDtypeStruct((1, _NPAD), jnp.float32)],
    )(degparts)
    din = din2.reshape(_NPAD)
    s_out = sout2.reshape(_NPAD, 1)

    fo_s, io_s = _stage_c(oidx3, iidx3, d2d)
    fo_t, io_t = _stage_c(iidx3, oidx3, d2d)
    ni0 = node_in[:1].reshape(1, 1)
    no0 = node_out[:1].reshape(1, 1)
    snf2, sns2 = _stage_c2(fo_s, io_s, ni0)
    tnf2, tns2 = _stage_c2(fo_t, io_t, no0)

    geo = _stage_d(oidx3, iidx3, d2d, px, py, pz,
                   snf2.reshape(_NPAD), sns2.reshape(_NPAD),
                   tnf2.reshape(_NPAD), tns2.reshape(_NPAD), din)

    msgA = pl.pallas_call(
        _edge_kernel,
        grid=(_E // _EBLK,),
        in_specs=[pl.BlockSpec((14, _EBLK), lambda i: (0, i)),
                  pl.BlockSpec((_MID, 12), lambda i: (0, 0)),
                  pl.BlockSpec((_MID, 6), lambda i: (0, 0)),
                  pl.BlockSpec((_MID, _D), lambda i: (0, 0)),
                  pl.BlockSpec((_MID + 2, _D), lambda i: (0, 0))],
        out_specs=pl.BlockSpec((_EBLK, _D), lambda i: (i, 0)),
        out_shape=jax.ShapeDtypeStruct((_E, _D), jnp.float32),
    )(geo, Wl1T, Wg1T, Wloc, Wglb2)

    nembP = pl.pallas_call(
        _nemb_kernel,
        grid=(_N // _NBLK,),
        in_specs=[pl.BlockSpec((_NBLK, _D), lambda i: (i, 0)),
                  pl.BlockSpec((_D, _D), lambda i: (0, 0)),
                  pl.BlockSpec((1, _D), lambda i: (0, 0)),
                  pl.BlockSpec((_NBLK, 1), lambda i: (i, 0))],
        out_specs=pl.BlockSpec((_NBLK, _D), lambda i: (i, 0)),
        out_shape=jax.ShapeDtypeStruct((_N, _D), jnp.float32),
    )(x, W_in, b_in[None, :], din[:_N].reshape(_N, 1))
    parts_n = _aggregate_stream(nembP, oidx3, iidx3, gather=True)


    parts_m = _aggregate_stream(msgA, oidx3, iidx3, gather=False)

    out1, stats = pl.pallas_call(
        _combine_kernel,
        grid=(_N // _NBLK,),
        in_specs=[pl.BlockSpec((2, _NBLK, _D), lambda i: (0, i, 0)),
                  pl.BlockSpec((2, _NBLK, _D), lambda i: (0, i, 0)),
                  pl.BlockSpec((_NBLK, 1), lambda i: (i, 0)),
                  pl.BlockSpec((_D, _D), lambda i: (0, 0)),
                  pl.BlockSpec((1, _D), lambda i: (0, 0))],
        out_specs=[pl.BlockSpec((_NBLK, _D), lambda i: (i, 0)),
                   pl.BlockSpec((2, _D), lambda i: (0, 0))],
        out_shape=[jax.ShapeDtypeStruct((_N, _D), jnp.float32),
                   jax.ShapeDtypeStruct((2, _D), jnp.float32)],
    )(parts_m, parts_n, s_out, Wm, bm[None, :])

    mean = stats[0] / _N
    var = stats[1] / _N - mean * mean
    scale = gamma / jnp.sqrt(var + 1e-5)
    shift = beta - mean * scale

    out = pl.pallas_call(
        _bn_kernel,
        grid=(_N // _NBLK,),
        in_specs=[pl.BlockSpec((_NBLK, _D), lambda i: (i, 0)),
                  pl.BlockSpec((1, _D), lambda i: (0, 0)),
                  pl.BlockSpec((1, _D), lambda i: (0, 0))],
        out_specs=pl.BlockSpec((_NBLK, _D), lambda i: (i, 0)),
        out_shape=jax.ShapeDtypeStruct((_N, _D), jnp.float32),
    )(out1, scale[None, :], shift[None, :])
    return out
